# baseline (speedup 1.0000x reference)
import functools
import math

import jax
import jax.numpy as jnp
from jax import lax
from jax.experimental import pallas as pl
from jax.experimental.pallas import tpu as pltpu
from jax.experimental.pallas import tpu_sc as plsc

N_NODES = 10000
N_EDGES = 160000
N_TRIP = 160000
N_GRAPHS = 512
H = 128
R = 6
S = 7
INT_EMB = 64
BD = 8
BA = 8
BT = 8
OUT_EMB = 128
OUT_DIM = 1
CUTOFF = 10.0
P_ENV = 5
NUM_LAYERS = 4


def _swish(x):
    return x * jax.nn.sigmoid(x)


_CS = 26752
_BUF = 26880
_TBE = 80
_CAP = 5000


def _part_body(ji_hbm, kj_hbm, kreg_hbm, treg_hbm, dreg_hbm, cnt_hbm,
               jb, kb, regk, regt, regd, cbuf):
    cid = lax.axis_index("c")
    sid = lax.axis_index("s")
    w = cid * 16 + sid

    def initrow(i, carry):
        regk[pl.ds(i * 16, 16)] = jnp.zeros((16,), jnp.int32)
        regt[pl.ds(i * 16, 16)] = jnp.zeros((16,), jnp.int32)
        regd[pl.ds(i * 16, 16)] = jnp.full((16,), _CS, jnp.int32)
        return carry
    lax.fori_loop(0, 6 * _CAP // 16, initrow, 0)

    lanes = lax.iota(jnp.int32, 16)

    def blk(b, cnts):
        base = w * _CAP + b * _TBE
        pltpu.sync_copy(ji_hbm.at[pl.ds(base, _TBE)], jb)
        pltpu.sync_copy(kj_hbm.at[pl.ds(base, _TBE)], kb)
        new = list(cnts)
        for k in range(_TBE // 16):
            jv = jb[pl.ds(k * 16, 16)]
            kv = kb[pl.ds(k * 16, 16)]
            tv = lanes + (base + k * 16)
            for c in range(6):
                loc = jv - c * _CS
                msk = (loc >= 0) & (loc < _CS)
                mi = msk.astype(jnp.int32)
                slot = c * _CAP + new[c] + plsc.cumsum(mi) - 1
                plsc.store_scatter(regk, [slot], kv, mask=msk)
                plsc.store_scatter(regt, [slot], tv, mask=msk)
                plsc.store_scatter(regd, [slot], loc, mask=msk)
                new[c] = new[c] + jnp.sum(mi)
        return tuple(new)
    z = jnp.int32(0)
    cnts = lax.fori_loop(0, _CAP // _TBE, blk, (z, z, z, z, z, z))

    cv = jnp.zeros((16,), jnp.int32)
    for c in range(6):
        cv = jnp.where(lanes == c, cnts[c], cv)
    cbuf[pl.ds(0, 16)] = cv
    pltpu.sync_copy(cbuf, cnt_hbm.at[w])
    pltpu.sync_copy(regk, kreg_hbm.at[w])
    pltpu.sync_copy(regt, treg_hbm.at[w])
    pltpu.sync_copy(regd, dreg_hbm.at[w])


_part_call = pl.kernel(
    _part_body,
    out_type=(
        jax.ShapeDtypeStruct((32, 6 * _CAP), jnp.int32),
        jax.ShapeDtypeStruct((32, 6 * _CAP), jnp.int32),
        jax.ShapeDtypeStruct((32, 6 * _CAP), jnp.int32),
        jax.ShapeDtypeStruct((32, 16), jnp.int32),
    ),
    mesh=plsc.VectorSubcoreMesh(core_axis_name="c", subcore_axis_name="s"),
    scratch_types=[
        pltpu.VMEM((_TBE,), jnp.int32),
        pltpu.VMEM((_TBE,), jnp.int32),
        pltpu.VMEM((6 * _CAP,), jnp.int32),
        pltpu.VMEM((6 * _CAP,), jnp.int32),
        pltpu.VMEM((6 * _CAP,), jnp.int32),
        pltpu.VMEM((16,), jnp.int32),
    ],
    compiler_params=pltpu.CompilerParams(use_tc_tiling_on_sc=False, needs_layout_passes=False),
)


def _agg_body(h_hbm, s_hbm, kreg_hbm, treg_hbm, dreg_hbm, cnt_hbm, out_hbm,
              kb, tb_, db, hv, sv, zbuf, cntv, sem, shared):
    cid = lax.axis_index("c")
    sid = lax.axis_index("s")

    def zrow(r, carry):
        for q in range(4):
            zbuf[r, pl.ds(q * 16, 16)] = jnp.zeros((16,), jnp.float32)
        return carry
    lax.fori_loop(0, _TBE, zrow, 0)

    def do_chunk(c, carry):
        chunk = cid * 3 + c
        cbase = chunk * _CS
        z0 = sid * 1680
        for t in range(21):
            pltpu.sync_copy(zbuf, shared.at[pl.ds(z0 + t * 80, 80)])
        plsc.subcore_barrier()

        def do_region(rr, carry2):
            w2 = sid * 2 + rr
            pltpu.sync_copy(cnt_hbm.at[w2], cntv)
            lanes = lax.iota(jnp.int32, 16)
            cnt = jnp.sum(jnp.where(lanes == chunk, cntv[pl.ds(0, 16)], 0))
            nb = (cnt + (_TBE - 1)) // _TBE

            def blk(b, carry3):
                o = chunk * _CAP + b * _TBE
                pltpu.sync_copy(kreg_hbm.at[w2, pl.ds(o, _TBE)], kb)
                pltpu.sync_copy(treg_hbm.at[w2, pl.ds(o, _TBE)], tb_)
                pltpu.sync_copy(dreg_hbm.at[w2, pl.ds(o, _TBE)], db)
                pltpu.async_copy(h_hbm.at[kb], hv, sem).wait()
                pltpu.async_copy(s_hbm.at[tb_], sv, sem).wait()

                def mrow(r, carry4):
                    for q in range(4):
                        hv[r, pl.ds(q * 16, 16)] = (
                            hv[r, pl.ds(q * 16, 16)] * sv[r, pl.ds(q * 16, 16)])
                    return carry4
                lax.fori_loop(0, _TBE, mrow, 0)
                pltpu.sync_copy(hv, shared.at[db], add=True)
                return carry3
            lax.fori_loop(0, nb, blk, 0)
            return carry2
        lax.fori_loop(0, 2, do_region, 0)
        plsc.subcore_barrier()
        d0 = sid * 1672
        pltpu.sync_copy(shared.at[pl.ds(d0, 1672)],
                        out_hbm.at[pl.ds(cbase + d0, 1672)])
        plsc.subcore_barrier()
        return carry
    lax.fori_loop(0, 3, do_chunk, 0)


_agg_call = pl.kernel(
    _agg_body,
    out_type=jax.ShapeDtypeStruct((6 * _CS, 64), jnp.float32),
    mesh=plsc.VectorSubcoreMesh(core_axis_name="c", subcore_axis_name="s"),
    scratch_types=[
        pltpu.VMEM((_TBE,), jnp.int32),
        pltpu.VMEM((_TBE,), jnp.int32),
        pltpu.VMEM((_TBE,), jnp.int32),
        pltpu.VMEM((_TBE, 64), jnp.float32),
        pltpu.VMEM((_TBE, 64), jnp.float32),
        pltpu.VMEM((80, 64), jnp.float32),
        pltpu.VMEM((16,), jnp.int32),
        pltpu.SemaphoreType.DMA,
        pltpu.VMEM_SHARED((_BUF, 64), jnp.float32),
    ],
    compiler_params=pltpu.CompilerParams(use_tc_tiling_on_sc=False, needs_layout_passes=False),
)


_NROWS = 10240


def _seg_nodes_body(e_hbm, i_hbm, out_hbm, iv, ev, zbuf, shared):
    cid = lax.axis_index("c")
    sid = lax.axis_index("s")

    def zrow(r, carry):
        for q in range(8):
            zbuf[r, pl.ds(q * 16, 16)] = jnp.zeros((16,), jnp.float32)
        return carry
    lax.fori_loop(0, 40, zrow, 0)
    z0 = sid * 640
    for t in range(16):
        pltpu.sync_copy(zbuf, shared.at[pl.ds(z0 + t * 40, 40)])
    plsc.subcore_barrier()
    w = cid * 16 + sid
    t0 = w * 5000

    def blk(b, carry):
        base = t0 + b * 40
        pltpu.sync_copy(i_hbm.at[pl.ds(base, 40)], iv)
        pltpu.sync_copy(e_hbm.at[pl.ds(base, 40)], ev)
        pltpu.sync_copy(ev, shared.at[iv], add=True)
        return carry
    lax.fori_loop(0, 125, blk, 0)
    plsc.subcore_barrier()
    pltpu.sync_copy(shared.at[pl.ds(z0, 640)], out_hbm.at[cid, pl.ds(z0, 640)])


_seg_nodes_call = pl.kernel(
    _seg_nodes_body,
    out_type=jax.ShapeDtypeStruct((2, _NROWS, 128), jnp.float32),
    mesh=plsc.VectorSubcoreMesh(core_axis_name="c", subcore_axis_name="s"),
    scratch_types=[
        pltpu.VMEM((40,), jnp.int32),
        pltpu.VMEM((40, 128), jnp.float32),
        pltpu.VMEM((40, 128), jnp.float32),
        pltpu.VMEM_SHARED((_NROWS, 128), jnp.float32),
    ],
    compiler_params=pltpu.CompilerParams(use_tc_tiling_on_sc=False, needs_layout_passes=False),
)


_TE = 1280


def _row(nr, nc):
    return pl.BlockSpec((nr, nc), lambda i: (i, 0))


def _full(shape):
    return pl.BlockSpec(shape, lambda i: tuple(0 for _ in shape))


def _tcf_body(e1_ref, rbf_ref, wji, bji, wkj, bkj, wrc, wdown, xji_ref, h_ref):
    e1 = e1_ref[...]
    xji_ref[...] = _swish(e1 @ wji[...] + bji[...])
    xkj = _swish(e1 @ wkj[...] + bkj[...]) * (rbf_ref[...] @ wrc[...])
    h_ref[...] = _swish(xkj @ wdown[...])


_tcf_call = pl.pallas_call(
    _tcf_body,
    grid=(N_EDGES // _TE,),
    in_specs=[_row(_TE, H), _row(_TE, 8), _full((H, H)), _full((1, H)),
              _full((H, H)), _full((1, H)), _full((8, H)), _full((H, 64))],
    out_specs=[_row(_TE, H), _row(_TE, 64)],
    out_shape=[jax.ShapeDtypeStruct((N_EDGES, H), jnp.float32),
               jax.ShapeDtypeStruct((N_EDGES, 64), jnp.float32)],
)


def _tcs_body(b42_ref, cm_ref, w1, wt, w2s, w2t, s_ref):
    b42 = b42_ref[...]
    cm = cm_ref[...]
    sb8 = b42 @ w1[...]
    tp = b42 @ wt[...]
    tb8 = cm[:, 0:1] * tp[:, 0:8]
    for m in range(1, S):
        tb8 = tb8 + cm[:, m:m + 1] * tp[:, 8 * m:8 * m + 8]
    s_ref[...] = (sb8 @ w2s[...]) * (tb8 @ w2t[...])


_tcs_call = pl.pallas_call(
    _tcs_body,
    grid=(N_TRIP // _TE,),
    in_specs=[_row(_TE, 48), _row(_TE, 8), _full((48, 8)), _full((48, 56)),
              _full((8, 64)), _full((8, 64))],
    out_specs=_row(_TE, 64),
    out_shape=jax.ShapeDtypeStruct((N_TRIP, 64), jnp.float32),
)


def _res_block(e, w1, b1, w2, b2):
    return e + _swish(_swish(e @ w1[...] + b1[...]) @ w2[...] + b2[...])


def _tcg_body(want_e2, agg_ref, xji_ref, e1o_ref, rbf_ref,
              wup, bw1, bb1, bw2, bb2, wmid, bmid,
              aw1, ab1, aw2, ab2, aw3, ab3, aw4, ab4, wrbf, *out_refs):
    e1n = xji_ref[...] + _swish(agg_ref[...] @ wup[...])
    e1n = _res_block(e1n, bw1, bb1, bw2, bb2)
    e1n = _swish(e1n @ wmid[...] + bmid[...]) + e1o_ref[...]
    e1n = _res_block(e1n, aw1, ab1, aw2, ab2)
    e1n = _res_block(e1n, aw3, ab3, aw4, ab4)
    out_refs[0][...] = e1n
    if want_e2:
        out_refs[1][...] = (rbf_ref[...] @ wrbf[...]) * e1n


def _make_tcg(want_e2):
    outs = [jax.ShapeDtypeStruct((N_EDGES, H), jnp.float32)]
    ospecs = [_row(_TE, H)]
    if want_e2:
        outs.append(jax.ShapeDtypeStruct((N_EDGES, H), jnp.float32))
        ospecs.append(_row(_TE, H))
    wspecs = ([_full((64, H))] + [_full((H, H)), _full((1, H))] * 2
              + [_full((H, H)), _full((1, H))] * 5 + [_full((8, H))])
    return pl.pallas_call(
        functools.partial(_tcg_body, want_e2),
        grid=(N_EDGES // _TE,),
        in_specs=[_row(_TE, 64), _row(_TE, H), _row(_TE, H), _row(_TE, 8)]
        + wspecs,
        out_specs=ospecs,
        out_shape=outs,
    )


_tcg_call = _make_tcg(False)
_tcg_e2_call = _make_tcg(True)


def _tci_body(xi_ref, xj_ref, rbf_ref, wr0, br0, w1, w2, w3, bcat, e1_ref):
    rbf0 = _swish(rbf_ref[...] @ wr0[...] + br0[...])
    e1_ref[...] = _swish(xi_ref[...] @ w1[...] + xj_ref[...] @ w2[...]
                         + rbf0 @ w3[...] + bcat[...])


_tci_call = pl.pallas_call(
    _tci_body,
    grid=(N_EDGES // _TE,),
    in_specs=[_row(_TE, H), _row(_TE, H), _row(_TE, 8), _full((8, H)),
              _full((1, H)), _full((H, H)), _full((H, H)), _full((H, H)),
              _full((1, H))],
    out_specs=_row(_TE, H),
    out_shape=jax.ShapeDtypeStruct((N_EDGES, H), jnp.float32),
)


_TV = 1024


def _tcv_body(va_ref, vb_ref, b_ref, wu, bu, l1w, l1b, l2w, l2b, wf, out_ref):
    v = _swish((va_ref[...] + vb_ref[...]) @ wu[...] + bu[...])
    v = _swish(v @ l1w[...] + l1b[...])
    v = _swish(v @ l2w[...] + l2b[...])
    v = v @ wf[...]
    onehot = (b_ref[...][:, None]
              == lax.broadcasted_iota(jnp.int32, (1, N_GRAPHS), 1)
              ).astype(jnp.float32)
    acc = lax.dot_general(v, onehot, (((0,), (0,)), ((), ())))

    @pl.when(pl.program_id(0) == 0)
    def _():
        out_ref[...] = jnp.zeros_like(out_ref)
    out_ref[...] += acc


_tcv_call = pl.pallas_call(
    _tcv_body,
    grid=(_NROWS // _TV,),
    in_specs=[_row(_TV, H), _row(_TV, H),
              pl.BlockSpec((_TV,), lambda i: (i,)), _full((H, OUT_EMB)),
              _full((1, OUT_EMB)), _full((OUT_EMB, OUT_EMB)),
              _full((1, OUT_EMB)), _full((OUT_EMB, OUT_EMB)),
              _full((1, OUT_EMB)), _full((OUT_EMB, OUT_DIM))],
    out_specs=pl.BlockSpec((1, N_GRAPHS), lambda i: (0, 0)),
    out_shape=jax.ShapeDtypeStruct((1, N_GRAPHS), jnp.float32),
)


def _envelope(x):
    p = P_ENV + 1
    a = -(p + 1) * (p + 2) / 2.0
    b = p * (p + 2)
    c = -p * (p + 1) / 2.0
    x4 = (x * x) * (x * x)
    return 1.0 / x + a * x4 * x + b * x4 * x * x + c * x4 * x * x * x


def _dist_emb(dist):
    x = jnp.clip(dist / CUTOFF, 1e-4, None)
    freqs = jnp.arange(1, R + 1, dtype=jnp.float32) * math.pi
    return _envelope(x)[:, None] * jnp.sin(freqs[None, :] * x[:, None])


def _sph_jl_all(x):
    out = []
    for l in range(S):
        z = jnp.clip(x[l], 0.1, None)
        sz = jnp.sin(z)
        cz = jnp.cos(z)
        j0 = sz / z
        if l == 0:
            out.append(j0)
            continue
        j1 = sz / (z * z) - cz / z
        jm, jc = j0, j1
        for ll in range(2, l + 1):
            jm, jc = jc, (2.0 * ll - 1.0) / z * jc - jm
        out.append(jc)
    return out


def _base42(dist_t, ct):
    x = jnp.clip(dist_t / CUTOFF, 1e-4, None)
    ps = [jnp.ones_like(ct), ct]
    for l in range(2, S):
        ps.append(((2.0 * l - 1.0) * ct * ps[l - 1] - (l - 1.0) * ps[l - 2]) / l)
    zs = []
    for l in range(S):
        roots = (jnp.arange(1, R + 1, dtype=jnp.float32) + 0.5 * l) * math.pi
        zs.append(roots[None, :] * x[:, None])
    jls = _sph_jl_all(zs)
    feats = [jls[l] * ps[l][:, None] for l in range(S)]
    return jnp.concatenate(feats, axis=1)


def _pad_rows(w, rows):
    return jnp.pad(w, ((0, rows - w.shape[0]), (0, 0)))


def _b2(b):
    return b.reshape(1, -1)


def _update_e(p, e1, rbf8, base48, cosm8, part, want_e2):
    wrc = _pad_rows(p['lin_rbf1']['w'], 8) @ p['lin_rbf2']['w']
    xji, h = _tcf_call(e1, rbf8, p['lin_ji']['w'], _b2(p['lin_ji']['b']),
                       p['lin_kj']['w'], _b2(p['lin_kj']['b']), wrc,
                       p['lin_down']['w'])
    w1 = p['lin_t1']['w'].reshape(S, S, R, BT).transpose(0, 2, 1, 3)
    w1 = _pad_rows(w1.reshape(S * R, S * BT), 48)
    s = _tcs_call(base48, cosm8, _pad_rows(p['lin_sbf1']['w'], 48), w1,
                  p['lin_sbf2']['w'], p['lin_t2']['w'])
    agg = _agg_call(h, s, *part)[:N_EDGES]
    (b1, b2), = p['before_skip']
    (a1, a2), (a3, a4) = p['after_skip']
    wrbf = _pad_rows(p['lin_rbf']['w'], 8)
    call = _tcg_e2_call if want_e2 else _tcg_call
    outs = call(agg, xji, e1, rbf8, p['lin_up']['w'],
                b1['w'], _b2(b1['b']), b2['w'], _b2(b2['b']),
                p['lin_mid']['w'], _b2(p['lin_mid']['b']),
                a1['w'], _b2(a1['b']), a2['w'], _b2(a2['b']),
                a3['w'], _b2(a3['b']), a4['w'], _b2(a4['b']), wrbf)
    if want_e2:
        return outs[0], outs[1]
    return outs[0], None


def kernel(atoms, pos, batch, edge_index, idx_kj, idx_ji, idx_t, params):
    j_idx = edge_index[0]
    i_idx = edge_index[1]
    vecs = pos[j_idx] - pos[i_idx]
    dist = jnp.sqrt(jnp.sum(vecs ** 2, axis=-1) + 1e-12)
    pos_ji = vecs[idx_ji]
    pos_kj = vecs[idx_kj]
    ref_v = vecs[idx_t]
    a = jnp.sum(pos_ji * pos_kj, axis=-1)
    n1 = jnp.cross(pos_ji, pos_kj)
    b = jnp.sqrt(jnp.sum(n1 ** 2, axis=-1) + 1e-12)
    ct = a / jnp.sqrt(a * a + b * b)
    n2 = jnp.cross(pos_ji, ref_v)
    dist_ji = jnp.sqrt(jnp.sum(pos_ji ** 2, axis=-1) + 1e-12)
    t_b = jnp.sum(jnp.cross(n1, n2) * pos_ji, axis=-1) / dist_ji + 1e-6
    t_a = jnp.sum(n1 * n2, axis=-1) + 1e-6
    cphi = t_a / jnp.sqrt(t_a * t_a + t_b * t_b + 1e-30)
    cs = [jnp.ones_like(cphi), cphi]
    for m in range(2, S):
        cs.append(2.0 * cphi * cs[m - 1] - cs[m - 2])
    cosm = jnp.stack(cs, axis=1)

    rbf = _dist_emb(dist)
    dist_t = jnp.sqrt(jnp.sum(pos_kj ** 2, axis=-1) + 1e-12)
    base42 = _base42(dist_t, ct)

    rbf8 = jnp.pad(rbf, ((0, 0), (0, 2)))
    base48 = jnp.pad(base42, ((0, 0), (0, 6)))
    cosm8 = jnp.pad(cosm, ((0, 0), (0, 1)))

    x = params['node_emb'][atoms]
    pi_ = params['init']
    wcat = pi_['lin']['w']
    e1 = _tci_call(x[i_idx], x[j_idx], rbf8, _pad_rows(pi_['rbf0']['w'], 8),
                   _b2(pi_['rbf0']['b']), wcat[:H], wcat[H:2 * H],
                   wcat[2 * H:], _b2(pi_['lin']['b']))

    part = _part_call(idx_ji.astype(jnp.int32), idx_kj.astype(jnp.int32))
    e2 = None
    for layer in range(NUM_LAYERS):
        e1, e2 = _update_e(params['update_es'][layer], e1, rbf8, base48,
                           cosm8, part, want_e2=(layer == NUM_LAYERS - 1))

    pv = params['update_vs'][NUM_LAYERS - 1]
    vp = _seg_nodes_call(e2, i_idx.astype(jnp.int32))
    bpad = jnp.concatenate([batch.astype(jnp.int32),
                            jnp.full((_NROWS - N_NODES,), N_GRAPHS, jnp.int32)])
    out = _tcv_call(vp[0], vp[1], bpad, pv['lin_up']['w'],
                    _b2(pv['lin_up']['b']), pv['lins'][0]['w'],
                    _b2(pv['lins'][0]['b']), pv['lins'][1]['w'],
                    _b2(pv['lins'][1]['b']), pv['lin']['w'])
    return out.reshape(N_GRAPHS, OUT_DIM)


# speedup vs baseline: 1.0027x; 1.0027x over previous
import functools
import math

import jax
import jax.numpy as jnp
from jax import lax
from jax.experimental import pallas as pl
from jax.experimental.pallas import tpu as pltpu
from jax.experimental.pallas import tpu_sc as plsc

N_NODES = 10000
N_EDGES = 160000
N_TRIP = 160000
N_GRAPHS = 512
H = 128
R = 6
S = 7
INT_EMB = 64
BD = 8
BA = 8
BT = 8
OUT_EMB = 128
OUT_DIM = 1
CUTOFF = 10.0
P_ENV = 5
NUM_LAYERS = 4


def _swish(x):
    return x * jax.nn.sigmoid(x)


_CS = 26752
_BUF = 26880
_TBE = 80
_CAP = 5120
_TPW = 5000


def _part_body(ji_hbm, kj_hbm, kreg_hbm, treg_hbm, dreg_hbm, cnt_hbm,
               jb, kb, regk, regt, regd, cbuf):
    cid = lax.axis_index("c")
    sid = lax.axis_index("s")
    w = cid * 16 + sid

    def initrow(i, carry):
        regk[pl.ds(i * 16, 16)] = jnp.zeros((16,), jnp.int32)
        regt[pl.ds(i * 16, 16)] = jnp.zeros((16,), jnp.int32)
        regd[pl.ds(i * 16, 16)] = jnp.full((16,), _CS, jnp.int32)
        return carry
    lax.fori_loop(0, 6 * _CAP // 16, initrow, 0)

    lanes = lax.iota(jnp.int32, 16)

    def group(jv, kv, tv, valid, cnts):
        new = list(cnts)
        for c in range(6):
            loc = jv - c * _CS
            msk = (loc >= 0) & (loc < _CS) & valid
            mi = msk.astype(jnp.int32)
            slot = c * _CAP + new[c] + plsc.cumsum(mi) - 1
            plsc.store_scatter(regk, [slot], kv, mask=msk)
            plsc.store_scatter(regt, [slot], tv, mask=msk)
            plsc.store_scatter(regd, [slot], loc, mask=msk)
            new[c] = new[c] + jnp.sum(mi)
        return tuple(new)

    full = lanes < 16

    def blk(b, cnts):
        base = w * _TPW + b * _TBE
        pltpu.sync_copy(ji_hbm.at[pl.ds(base, _TBE)], jb)
        pltpu.sync_copy(kj_hbm.at[pl.ds(base, _TBE)], kb)
        for k in range(_TBE // 16):
            cnts = group(jb[pl.ds(k * 16, 16)], kb[pl.ds(k * 16, 16)],
                         lanes + (base + k * 16), full, cnts)
        return cnts
    z = jnp.int32(0)
    cnts = lax.fori_loop(0, _TPW // _TBE, blk, (z, z, z, z, z, z))
    tbase = w * _TPW + (_TPW // _TBE) * _TBE
    pltpu.sync_copy(ji_hbm.at[pl.ds(tbase, 40)], jb.at[pl.ds(0, 40)])
    pltpu.sync_copy(kj_hbm.at[pl.ds(tbase, 40)], kb.at[pl.ds(0, 40)])
    for k in range(3):
        valid = full if k < 2 else (lanes < 8)
        cnts = group(jb[pl.ds(k * 16, 16)], kb[pl.ds(k * 16, 16)],
                     lanes + (tbase + k * 16), valid, cnts)

    cv = jnp.zeros((16,), jnp.int32)
    for c in range(6):
        cv = jnp.where(lanes == c, cnts[c], cv)
    cbuf[pl.ds(0, 16)] = cv
    pltpu.sync_copy(cbuf, cnt_hbm.at[w])
    pltpu.sync_copy(regk, kreg_hbm.at[w])
    pltpu.sync_copy(regt, treg_hbm.at[w])
    pltpu.sync_copy(regd, dreg_hbm.at[w])


_part_call = pl.kernel(
    _part_body,
    out_type=(
        jax.ShapeDtypeStruct((32, 6 * _CAP), jnp.int32),
        jax.ShapeDtypeStruct((32, 6 * _CAP), jnp.int32),
        jax.ShapeDtypeStruct((32, 6 * _CAP), jnp.int32),
        jax.ShapeDtypeStruct((32, 16), jnp.int32),
    ),
    mesh=plsc.VectorSubcoreMesh(core_axis_name="c", subcore_axis_name="s"),
    scratch_types=[
        pltpu.VMEM((_TBE,), jnp.int32),
        pltpu.VMEM((_TBE,), jnp.int32),
        pltpu.VMEM((6 * _CAP,), jnp.int32),
        pltpu.VMEM((6 * _CAP,), jnp.int32),
        pltpu.VMEM((6 * _CAP,), jnp.int32),
        pltpu.VMEM((16,), jnp.int32),
    ],
    compiler_params=pltpu.CompilerParams(use_tc_tiling_on_sc=False, needs_layout_passes=False),
)


def _agg_body(h_hbm, s_hbm, kreg_hbm, treg_hbm, dreg_hbm, cnt_hbm, out_hbm,
              kb, tb_, db, hv, sv, zbuf, cntv, sem, shared):
    cid = lax.axis_index("c")
    sid = lax.axis_index("s")

    def zrow(r, carry):
        for q in range(4):
            zbuf[r, pl.ds(q * 16, 16)] = jnp.zeros((16,), jnp.float32)
        return carry
    lax.fori_loop(0, _TBE, zrow, 0)

    def do_chunk(c, carry):
        chunk = cid * 3 + c
        cbase = chunk * _CS
        z0 = sid * 1680
        for t in range(21):
            pltpu.sync_copy(zbuf, shared.at[pl.ds(z0 + t * 80, 80)])
        plsc.subcore_barrier()

        def do_region(rr, carry2):
            w2 = sid * 2 + rr
            pltpu.sync_copy(cnt_hbm.at[w2], cntv)
            lanes = lax.iota(jnp.int32, 16)
            cnt = jnp.sum(jnp.where(lanes == chunk, cntv[pl.ds(0, 16)], 0))
            nb = (cnt + 127) // 128

            def blk(b, carry3):
                o = chunk * _CAP + b * 128
                pltpu.sync_copy(kreg_hbm.at[w2, pl.ds(o, 128)], kb)
                pltpu.sync_copy(treg_hbm.at[w2, pl.ds(o, 128)], tb_)
                pltpu.sync_copy(dreg_hbm.at[w2, pl.ds(o, 128)], db)
                cph = pltpu.async_copy(h_hbm.at[kb], hv, sem)
                cps = pltpu.async_copy(s_hbm.at[tb_], sv, sem)
                cph.wait()
                cps.wait()

                def mrow(r, carry4):
                    for q in range(4):
                        hv[r, pl.ds(q * 16, 16)] = (
                            hv[r, pl.ds(q * 16, 16)] * sv[r, pl.ds(q * 16, 16)])
                    return carry4
                lax.fori_loop(0, 128, mrow, 0)
                pltpu.sync_copy(hv, shared.at[db], add=True)
                return carry3
            lax.fori_loop(0, nb, blk, 0)
            return carry2
        lax.fori_loop(0, 2, do_region, 0)
        plsc.subcore_barrier()
        d0 = sid * 1672
        pltpu.sync_copy(shared.at[pl.ds(d0, 1672)],
                        out_hbm.at[pl.ds(cbase + d0, 1672)])
        plsc.subcore_barrier()
        return carry
    lax.fori_loop(0, 3, do_chunk, 0)


_agg_call = pl.kernel(
    _agg_body,
    out_type=jax.ShapeDtypeStruct((6 * _CS, 64), jnp.float32),
    mesh=plsc.VectorSubcoreMesh(core_axis_name="c", subcore_axis_name="s"),
    scratch_types=[
        pltpu.VMEM((128,), jnp.int32),
        pltpu.VMEM((128,), jnp.int32),
        pltpu.VMEM((128,), jnp.int32),
        pltpu.VMEM((128, 64), jnp.float32),
        pltpu.VMEM((128, 64), jnp.float32),
        pltpu.VMEM((80, 64), jnp.float32),
        pltpu.VMEM((16,), jnp.int32),
        pltpu.SemaphoreType.DMA,
        pltpu.VMEM_SHARED((_BUF, 64), jnp.float32),
    ],
    compiler_params=pltpu.CompilerParams(use_tc_tiling_on_sc=False, needs_layout_passes=False),
)


_NROWS = 10240


def _seg_nodes_body(e_hbm, i_hbm, out_hbm, iv, ev, zbuf, shared):
    cid = lax.axis_index("c")
    sid = lax.axis_index("s")

    def zrow(r, carry):
        for q in range(8):
            zbuf[r, pl.ds(q * 16, 16)] = jnp.zeros((16,), jnp.float32)
        return carry
    lax.fori_loop(0, 40, zrow, 0)
    z0 = sid * 640
    for t in range(16):
        pltpu.sync_copy(zbuf, shared.at[pl.ds(z0 + t * 40, 40)])
    plsc.subcore_barrier()
    w = cid * 16 + sid
    t0 = w * 5000

    def blk(b, carry):
        base = t0 + b * 40
        pltpu.sync_copy(i_hbm.at[pl.ds(base, 40)], iv)
        pltpu.sync_copy(e_hbm.at[pl.ds(base, 40)], ev)
        pltpu.sync_copy(ev, shared.at[iv], add=True)
        return carry
    lax.fori_loop(0, 125, blk, 0)
    plsc.subcore_barrier()
    pltpu.sync_copy(shared.at[pl.ds(z0, 640)], out_hbm.at[cid, pl.ds(z0, 640)])


_seg_nodes_call = pl.kernel(
    _seg_nodes_body,
    out_type=jax.ShapeDtypeStruct((2, _NROWS, 128), jnp.float32),
    mesh=plsc.VectorSubcoreMesh(core_axis_name="c", subcore_axis_name="s"),
    scratch_types=[
        pltpu.VMEM((40,), jnp.int32),
        pltpu.VMEM((40, 128), jnp.float32),
        pltpu.VMEM((40, 128), jnp.float32),
        pltpu.VMEM_SHARED((_NROWS, 128), jnp.float32),
    ],
    compiler_params=pltpu.CompilerParams(use_tc_tiling_on_sc=False, needs_layout_passes=False),
)


_TE = 1280


def _row(nr, nc):
    return pl.BlockSpec((nr, nc), lambda i: (i, 0))


def _full(shape):
    return pl.BlockSpec(shape, lambda i: tuple(0 for _ in shape))


def _mm(x, w):
    return jnp.dot(x, w, preferred_element_type=jnp.float32)


def _tcf_body(e1_ref, rbf_ref, wji, bji, wkj, bkj, wrc, wdown, xji_ref, h_ref):
    e1 = e1_ref[...]
    xji_ref[...] = _swish(_mm(e1, wji[...]) + bji[...])
    xkj = _swish(_mm(e1, wkj[...]) + bkj[...]) * _mm(rbf_ref[...], wrc[...])
    h_ref[...] = _swish(_mm(xkj, wdown[...]))


_tcf_call = pl.pallas_call(
    _tcf_body,
    grid=(N_EDGES // _TE,),
    in_specs=[_row(_TE, H), _row(_TE, 8), _full((H, H)), _full((1, H)),
              _full((H, H)), _full((1, H)), _full((8, H)), _full((H, 64))],
    out_specs=[_row(_TE, H), _row(_TE, 64)],
    out_shape=[jax.ShapeDtypeStruct((N_EDGES, H), jnp.float32),
               jax.ShapeDtypeStruct((N_EDGES, 64), jnp.float32)],
)


def _tcs_body(b42_ref, cm_ref, w1, wt, w2s, w2t, s_ref):
    b42 = b42_ref[...]
    cm = cm_ref[...]
    sb8 = _mm(b42, w1[...])
    tp = _mm(b42, wt[...])
    tb8 = cm[:, 0:1] * tp[:, 0:8]
    for m in range(1, S):
        tb8 = tb8 + cm[:, m:m + 1] * tp[:, 8 * m:8 * m + 8]
    s_ref[...] = _mm(sb8, w2s[...]) * _mm(tb8, w2t[...])


_tcs_call = pl.pallas_call(
    _tcs_body,
    grid=(N_TRIP // _TE,),
    in_specs=[_row(_TE, 48), _row(_TE, 8), _full((48, 8)), _full((48, 56)),
              _full((8, 64)), _full((8, 64))],
    out_specs=_row(_TE, 64),
    out_shape=jax.ShapeDtypeStruct((N_TRIP, 64), jnp.float32),
)


def _res_block(e, w1, b1, w2, b2):
    return e + _swish(_mm(_swish(_mm(e, w1[...]) + b1[...]), w2[...]) + b2[...])


def _tcg_body(want_e2, agg_ref, xji_ref, e1o_ref, rbf_ref,
              wup, bw1, bb1, bw2, bb2, wmid, bmid,
              aw1, ab1, aw2, ab2, aw3, ab3, aw4, ab4, wrbf, *out_refs):
    e1n = xji_ref[...] + _swish(_mm(agg_ref[...], wup[...]))
    e1n = _res_block(e1n, bw1, bb1, bw2, bb2)
    e1n = _swish(_mm(e1n, wmid[...]) + bmid[...]) + e1o_ref[...]
    e1n = _res_block(e1n, aw1, ab1, aw2, ab2)
    e1n = _res_block(e1n, aw3, ab3, aw4, ab4)
    out_refs[0][...] = e1n
    if want_e2:
        out_refs[1][...] = _mm(rbf_ref[...], wrbf[...]) * e1n


def _make_tcg(want_e2):
    outs = [jax.ShapeDtypeStruct((N_EDGES, H), jnp.float32)]
    ospecs = [_row(_TE, H)]
    if want_e2:
        outs.append(jax.ShapeDtypeStruct((N_EDGES, H), jnp.float32))
        ospecs.append(_row(_TE, H))
    wspecs = ([_full((64, H))] + [_full((H, H)), _full((1, H))] * 2
              + [_full((H, H)), _full((1, H))] * 5 + [_full((8, H))])
    return pl.pallas_call(
        functools.partial(_tcg_body, want_e2),
        grid=(N_EDGES // _TE,),
        in_specs=[_row(_TE, 64), _row(_TE, H), _row(_TE, H), _row(_TE, 8)]
        + wspecs,
        out_specs=ospecs,
        out_shape=outs,
    )


_tcg_call = _make_tcg(False)
_tcg_e2_call = _make_tcg(True)


def _tci_body(xi_ref, xj_ref, rbf_ref, wr0, br0, w1, w2, w3, bcat, e1_ref):
    rbf0 = _swish(_mm(rbf_ref[...], wr0[...]) + br0[...])
    e1_ref[...] = _swish(_mm(xi_ref[...], w1[...]) + _mm(xj_ref[...], w2[...])
                         + _mm(rbf0, w3[...]) + bcat[...])


_tci_call = pl.pallas_call(
    _tci_body,
    grid=(N_EDGES // _TE,),
    in_specs=[_row(_TE, H), _row(_TE, H), _row(_TE, 8), _full((8, H)),
              _full((1, H)), _full((H, H)), _full((H, H)), _full((H, H)),
              _full((1, H))],
    out_specs=_row(_TE, H),
    out_shape=jax.ShapeDtypeStruct((N_EDGES, H), jnp.float32),
)


_TV = 1024


def _tcv_body(va_ref, vb_ref, b_ref, wu, bu, l1w, l1b, l2w, l2b, wf, out_ref):
    v = _swish(_mm(va_ref[...] + vb_ref[...], wu[...]) + bu[...])
    v = _swish(_mm(v, l1w[...]) + l1b[...])
    v = _swish(_mm(v, l2w[...]) + l2b[...])
    v = v @ wf[...]
    onehot = (b_ref[...][:, None]
              == lax.broadcasted_iota(jnp.int32, (1, N_GRAPHS), 1)
              ).astype(jnp.float32)
    acc = lax.dot_general(v, onehot, (((0,), (0,)), ((), ())))

    @pl.when(pl.program_id(0) == 0)
    def _():
        out_ref[...] = jnp.zeros_like(out_ref)
    out_ref[...] += acc


_tcv_call = pl.pallas_call(
    _tcv_body,
    grid=(_NROWS // _TV,),
    in_specs=[_row(_TV, H), _row(_TV, H),
              pl.BlockSpec((_TV,), lambda i: (i,)), _full((H, OUT_EMB)),
              _full((1, OUT_EMB)), _full((OUT_EMB, OUT_EMB)),
              _full((1, OUT_EMB)), _full((OUT_EMB, OUT_EMB)),
              _full((1, OUT_EMB)), _full((OUT_EMB, OUT_DIM))],
    out_specs=pl.BlockSpec((1, N_GRAPHS), lambda i: (0, 0)),
    out_shape=jax.ShapeDtypeStruct((1, N_GRAPHS), jnp.float32),
)


def _envelope(x):
    p = P_ENV + 1
    a = -(p + 1) * (p + 2) / 2.0
    b = p * (p + 2)
    c = -p * (p + 1) / 2.0
    x4 = (x * x) * (x * x)
    return 1.0 / x + a * x4 * x + b * x4 * x * x + c * x4 * x * x * x


def _dist_emb(dist):
    x = jnp.clip(dist / CUTOFF, 1e-4, None)
    freqs = jnp.arange(1, R + 1, dtype=jnp.float32) * math.pi
    return _envelope(x)[:, None] * jnp.sin(freqs[None, :] * x[:, None])


def _sph_jl_all(x):
    out = []
    for l in range(S):
        z = jnp.clip(x[l], 0.1, None)
        sz = jnp.sin(z)
        cz = jnp.cos(z)
        j0 = sz / z
        if l == 0:
            out.append(j0)
            continue
        j1 = sz / (z * z) - cz / z
        jm, jc = j0, j1
        for ll in range(2, l + 1):
            jm, jc = jc, (2.0 * ll - 1.0) / z * jc - jm
        out.append(jc)
    return out


def _base42(dist_t, ct):
    x = jnp.clip(dist_t / CUTOFF, 1e-4, None)
    ps = [jnp.ones_like(ct), ct]
    for l in range(2, S):
        ps.append(((2.0 * l - 1.0) * ct * ps[l - 1] - (l - 1.0) * ps[l - 2]) / l)
    zs = []
    for l in range(S):
        roots = (jnp.arange(1, R + 1, dtype=jnp.float32) + 0.5 * l) * math.pi
        zs.append(roots[None, :] * x[:, None])
    jls = _sph_jl_all(zs)
    feats = [jls[l] * ps[l][:, None] for l in range(S)]
    return jnp.concatenate(feats, axis=1)


def _pad_rows(w, rows):
    return jnp.pad(w, ((0, rows - w.shape[0]), (0, 0)))


def _b2(b):
    return b.reshape(1, -1)


def _update_e(p, e1, rbf8, base48, cosm8, part, want_e2):
    wrc = _pad_rows(p['lin_rbf1']['w'], 8) @ p['lin_rbf2']['w']
    xji, h = _tcf_call(e1, rbf8, p['lin_ji']['w'], _b2(p['lin_ji']['b']),
                       p['lin_kj']['w'], _b2(p['lin_kj']['b']), wrc,
                       p['lin_down']['w'])
    w1 = p['lin_t1']['w'].reshape(S, S, R, BT).transpose(0, 2, 1, 3)
    w1 = _pad_rows(w1.reshape(S * R, S * BT), 48)
    s = _tcs_call(base48, cosm8, _pad_rows(p['lin_sbf1']['w'], 48), w1,
                  p['lin_sbf2']['w'], p['lin_t2']['w'])
    agg = _agg_call(h, s, *part)[:N_EDGES]
    (b1, b2), = p['before_skip']
    (a1, a2), (a3, a4) = p['after_skip']
    wrbf = _pad_rows(p['lin_rbf']['w'], 8)
    call = _tcg_e2_call if want_e2 else _tcg_call
    outs = call(agg, xji, e1, rbf8, p['lin_up']['w'],
                b1['w'], _b2(b1['b']), b2['w'], _b2(b2['b']),
                p['lin_mid']['w'], _b2(p['lin_mid']['b']),
                a1['w'], _b2(a1['b']), a2['w'], _b2(a2['b']),
                a3['w'], _b2(a3['b']), a4['w'], _b2(a4['b']), wrbf)
    if want_e2:
        return outs[0], outs[1]
    return outs[0], None


def kernel(atoms, pos, batch, edge_index, idx_kj, idx_ji, idx_t, params):
    j_idx = edge_index[0]
    i_idx = edge_index[1]
    vecs = pos[j_idx] - pos[i_idx]
    dist = jnp.sqrt(jnp.sum(vecs ** 2, axis=-1) + 1e-12)
    pos_ji = vecs[idx_ji]
    pos_kj = vecs[idx_kj]
    ref_v = vecs[idx_t]
    a = jnp.sum(pos_ji * pos_kj, axis=-1)
    n1 = jnp.cross(pos_ji, pos_kj)
    b = jnp.sqrt(jnp.sum(n1 ** 2, axis=-1) + 1e-12)
    ct = a / jnp.sqrt(a * a + b * b)
    n2 = jnp.cross(pos_ji, ref_v)
    dist_ji = jnp.sqrt(jnp.sum(pos_ji ** 2, axis=-1) + 1e-12)
    t_b = jnp.sum(jnp.cross(n1, n2) * pos_ji, axis=-1) / dist_ji + 1e-6
    t_a = jnp.sum(n1 * n2, axis=-1) + 1e-6
    cphi = t_a / jnp.sqrt(t_a * t_a + t_b * t_b + 1e-30)
    cs = [jnp.ones_like(cphi), cphi]
    for m in range(2, S):
        cs.append(2.0 * cphi * cs[m - 1] - cs[m - 2])
    cosm = jnp.stack(cs, axis=1)

    rbf = _dist_emb(dist)
    dist_t = jnp.sqrt(jnp.sum(pos_kj ** 2, axis=-1) + 1e-12)
    base42 = _base42(dist_t, ct)

    rbf8 = jnp.pad(rbf, ((0, 0), (0, 2)))
    base48 = jnp.pad(base42, ((0, 0), (0, 6)))
    cosm8 = jnp.pad(cosm, ((0, 0), (0, 1)))

    x = params['node_emb'][atoms]
    pi_ = params['init']
    wcat = pi_['lin']['w']
    e1 = _tci_call(x[i_idx], x[j_idx], rbf8, _pad_rows(pi_['rbf0']['w'], 8),
                   _b2(pi_['rbf0']['b']), wcat[:H], wcat[H:2 * H],
                   wcat[2 * H:], _b2(pi_['lin']['b']))

    part = _part_call(idx_ji.astype(jnp.int32), idx_kj.astype(jnp.int32))
    e2 = None
    for layer in range(NUM_LAYERS):
        e1, e2 = _update_e(params['update_es'][layer], e1, rbf8, base48,
                           cosm8, part, want_e2=(layer == NUM_LAYERS - 1))

    pv = params['update_vs'][NUM_LAYERS - 1]
    vp = _seg_nodes_call(e2, i_idx.astype(jnp.int32))
    bpad = jnp.concatenate([batch.astype(jnp.int32),
                            jnp.full((_NROWS - N_NODES,), N_GRAPHS, jnp.int32)])
    out = _tcv_call(vp[0], vp[1], bpad, pv['lin_up']['w'],
                    _b2(pv['lin_up']['b']), pv['lins'][0]['w'],
                    _b2(pv['lins'][0]['b']), pv['lins'][1]['w'],
                    _b2(pv['lins'][1]['b']), pv['lin']['w'])
    return out.reshape(N_GRAPHS, OUT_DIM)


# speedup vs baseline: 1.0302x; 1.0275x over previous
import functools
import math

import jax
import jax.numpy as jnp
from jax import lax
from jax.experimental import pallas as pl
from jax.experimental.pallas import tpu as pltpu
from jax.experimental.pallas import tpu_sc as plsc

N_NODES = 10000
N_EDGES = 160000
N_TRIP = 160000
N_GRAPHS = 512
H = 128
R = 6
S = 7
INT_EMB = 64
BD = 8
BA = 8
BT = 8
OUT_EMB = 128
OUT_DIM = 1
CUTOFF = 10.0
P_ENV = 5
NUM_LAYERS = 4


def _swish(x):
    return x * jax.nn.sigmoid(x)


_CS = 26752
_BUF = 26880
_TBE = 80
_CAP = 5120
_TPW = 5000


def _part_body(ji_hbm, kj_hbm, kreg_hbm, treg_hbm, dreg_hbm, cnt_hbm,
               jb, kb, regk, regt, regd, cbuf):
    cid = lax.axis_index("c")
    sid = lax.axis_index("s")
    w = cid * 16 + sid

    def initrow(i, carry):
        regk[pl.ds(i * 16, 16)] = jnp.zeros((16,), jnp.int32)
        regt[pl.ds(i * 16, 16)] = jnp.zeros((16,), jnp.int32)
        regd[pl.ds(i * 16, 16)] = jnp.full((16,), _CS, jnp.int32)
        return carry
    lax.fori_loop(0, 6 * _CAP // 16, initrow, 0)

    lanes = lax.iota(jnp.int32, 16)

    def group(jv, kv, tv, valid, cnts):
        new = list(cnts)
        for c in range(6):
            loc = jv - c * _CS
            msk = (loc >= 0) & (loc < _CS) & valid
            mi = msk.astype(jnp.int32)
            slot = c * _CAP + new[c] + plsc.cumsum(mi) - 1
            plsc.store_scatter(regk, [slot], kv, mask=msk)
            plsc.store_scatter(regt, [slot], tv, mask=msk)
            plsc.store_scatter(regd, [slot], loc, mask=msk)
            new[c] = new[c] + jnp.sum(mi)
        return tuple(new)

    full = lanes < 16

    def blk(b, cnts):
        base = w * _TPW + b * _TBE
        pltpu.sync_copy(ji_hbm.at[pl.ds(base, _TBE)], jb)
        pltpu.sync_copy(kj_hbm.at[pl.ds(base, _TBE)], kb)
        for k in range(_TBE // 16):
            cnts = group(jb[pl.ds(k * 16, 16)], kb[pl.ds(k * 16, 16)],
                         lanes + (base + k * 16), full, cnts)
        return cnts
    z = jnp.int32(0)
    cnts = lax.fori_loop(0, _TPW // _TBE, blk, (z, z, z, z, z, z))
    tbase = w * _TPW + (_TPW // _TBE) * _TBE
    pltpu.sync_copy(ji_hbm.at[pl.ds(tbase, 40)], jb.at[pl.ds(0, 40)])
    pltpu.sync_copy(kj_hbm.at[pl.ds(tbase, 40)], kb.at[pl.ds(0, 40)])
    for k in range(3):
        valid = full if k < 2 else (lanes < 8)
        cnts = group(jb[pl.ds(k * 16, 16)], kb[pl.ds(k * 16, 16)],
                     lanes + (tbase + k * 16), valid, cnts)

    cv = jnp.zeros((16,), jnp.int32)
    for c in range(6):
        cv = jnp.where(lanes == c, cnts[c], cv)
    cbuf[pl.ds(0, 16)] = cv
    pltpu.sync_copy(cbuf, cnt_hbm.at[w])
    pltpu.sync_copy(regk, kreg_hbm.at[w])
    pltpu.sync_copy(regt, treg_hbm.at[w])
    pltpu.sync_copy(regd, dreg_hbm.at[w])


_part_call = pl.kernel(
    _part_body,
    out_type=(
        jax.ShapeDtypeStruct((32, 6 * _CAP), jnp.int32),
        jax.ShapeDtypeStruct((32, 6 * _CAP), jnp.int32),
        jax.ShapeDtypeStruct((32, 6 * _CAP), jnp.int32),
        jax.ShapeDtypeStruct((32, 16), jnp.int32),
    ),
    mesh=plsc.VectorSubcoreMesh(core_axis_name="c", subcore_axis_name="s"),
    scratch_types=[
        pltpu.VMEM((_TBE,), jnp.int32),
        pltpu.VMEM((_TBE,), jnp.int32),
        pltpu.VMEM((6 * _CAP,), jnp.int32),
        pltpu.VMEM((6 * _CAP,), jnp.int32),
        pltpu.VMEM((6 * _CAP,), jnp.int32),
        pltpu.VMEM((16,), jnp.int32),
    ],
    compiler_params=pltpu.CompilerParams(use_tc_tiling_on_sc=False, needs_layout_passes=False),
)


def _agg_body(h_hbm, s_hbm, kreg_hbm, treg_hbm, dreg_hbm, cnt_hbm, out_hbm,
              kb, tb_, db, hv, sv, zbuf, cntv, sem, shared):
    cid = lax.axis_index("c")
    sid = lax.axis_index("s")

    def zrow(r, carry):
        for q in range(4):
            zbuf[r, pl.ds(q * 16, 16)] = jnp.zeros((16,), jnp.float32)
        return carry
    lax.fori_loop(0, _TBE, zrow, 0)

    def do_chunk(c, carry):
        chunk = cid * 3 + c
        cbase = chunk * _CS
        z0 = sid * 1680
        for t in range(21):
            pltpu.sync_copy(zbuf, shared.at[pl.ds(z0 + t * 80, 80)])
        plsc.subcore_barrier()

        def do_region(rr, carry2):
            w2 = sid * 2 + rr
            pltpu.sync_copy(cnt_hbm.at[w2], cntv)
            lanes = lax.iota(jnp.int32, 16)
            cnt = jnp.sum(jnp.where(lanes == chunk, cntv[pl.ds(0, 16)], 0))
            nb = (cnt + 127) // 128

            def blk(b, carry3):
                o = chunk * _CAP + b * 128
                pltpu.sync_copy(kreg_hbm.at[w2, pl.ds(o, 128)], kb)
                pltpu.sync_copy(treg_hbm.at[w2, pl.ds(o, 128)], tb_)
                pltpu.sync_copy(dreg_hbm.at[w2, pl.ds(o, 128)], db)
                cph = pltpu.async_copy(h_hbm.at[kb], hv, sem)
                cps = pltpu.async_copy(s_hbm.at[tb_], sv, sem)
                cph.wait()
                cps.wait()

                def mrow(r, carry4):
                    for q in range(4):
                        hv[r, pl.ds(q * 16, 16)] = (
                            hv[r, pl.ds(q * 16, 16)] * sv[r, pl.ds(q * 16, 16)])
                    return carry4
                lax.fori_loop(0, 128, mrow, 0)
                pltpu.sync_copy(hv, shared.at[db], add=True)
                return carry3
            lax.fori_loop(0, nb, blk, 0)
            return carry2
        lax.fori_loop(0, 2, do_region, 0)
        plsc.subcore_barrier()
        d0 = sid * 1672
        pltpu.sync_copy(shared.at[pl.ds(d0, 1672)],
                        out_hbm.at[pl.ds(cbase + d0, 1672)])
        plsc.subcore_barrier()
        return carry
    lax.fori_loop(0, 3, do_chunk, 0)


_agg_call = pl.kernel(
    _agg_body,
    out_type=jax.ShapeDtypeStruct((6 * _CS, 64), jnp.float32),
    mesh=plsc.VectorSubcoreMesh(core_axis_name="c", subcore_axis_name="s"),
    scratch_types=[
        pltpu.VMEM((128,), jnp.int32),
        pltpu.VMEM((128,), jnp.int32),
        pltpu.VMEM((128,), jnp.int32),
        pltpu.VMEM((128, 64), jnp.float32),
        pltpu.VMEM((128, 64), jnp.float32),
        pltpu.VMEM((80, 64), jnp.float32),
        pltpu.VMEM((16,), jnp.int32),
        pltpu.SemaphoreType.DMA,
        pltpu.VMEM_SHARED((_BUF, 64), jnp.float32),
    ],
    compiler_params=pltpu.CompilerParams(use_tc_tiling_on_sc=False, needs_layout_passes=False),
)


_NROWS = 10240


def _seg_nodes_body(e_hbm, i_hbm, out_hbm, iv, ev, zbuf, shared):
    cid = lax.axis_index("c")
    sid = lax.axis_index("s")

    def zrow(r, carry):
        for q in range(8):
            zbuf[r, pl.ds(q * 16, 16)] = jnp.zeros((16,), jnp.float32)
        return carry
    lax.fori_loop(0, 40, zrow, 0)
    z0 = sid * 640
    for t in range(16):
        pltpu.sync_copy(zbuf, shared.at[pl.ds(z0 + t * 40, 40)])
    plsc.subcore_barrier()
    w = cid * 16 + sid
    t0 = w * 5000

    def blk(b, carry):
        base = t0 + b * 40
        pltpu.sync_copy(i_hbm.at[pl.ds(base, 40)], iv)
        pltpu.sync_copy(e_hbm.at[pl.ds(base, 40)], ev)
        pltpu.sync_copy(ev, shared.at[iv], add=True)
        return carry
    lax.fori_loop(0, 125, blk, 0)
    plsc.subcore_barrier()
    pltpu.sync_copy(shared.at[pl.ds(z0, 640)], out_hbm.at[cid, pl.ds(z0, 640)])


_seg_nodes_call = pl.kernel(
    _seg_nodes_body,
    out_type=jax.ShapeDtypeStruct((2, _NROWS, 128), jnp.float32),
    mesh=plsc.VectorSubcoreMesh(core_axis_name="c", subcore_axis_name="s"),
    scratch_types=[
        pltpu.VMEM((40,), jnp.int32),
        pltpu.VMEM((40, 128), jnp.float32),
        pltpu.VMEM((40, 128), jnp.float32),
        pltpu.VMEM_SHARED((_NROWS, 128), jnp.float32),
    ],
    compiler_params=pltpu.CompilerParams(use_tc_tiling_on_sc=False, needs_layout_passes=False),
)


_TE = 2000


def _row(nr, nc):
    return pl.BlockSpec((nr, nc), lambda i: (i, 0))


def _full(shape):
    return pl.BlockSpec(shape, lambda i: tuple(0 for _ in shape))


def _mm(x, w):
    return jnp.dot(x, w, preferred_element_type=jnp.float32)


def _tcf_body(e1_ref, rbf_ref, wji, bji, wkj, bkj, wrc, wdown, xji_ref, h_ref):
    e1 = e1_ref[...]
    xji_ref[...] = _swish(_mm(e1, wji[...]) + bji[...])
    xkj = _swish(_mm(e1, wkj[...]) + bkj[...]) * _mm(rbf_ref[...], wrc[...])
    h_ref[...] = _swish(_mm(xkj, wdown[...]))


_tcf_call = pl.pallas_call(
    _tcf_body,
    grid=(N_EDGES // _TE,),
    in_specs=[_row(_TE, H), _row(_TE, 8), _full((H, H)), _full((1, H)),
              _full((H, H)), _full((1, H)), _full((8, H)), _full((H, 64))],
    out_specs=[_row(_TE, H), _row(_TE, 64)],
    out_shape=[jax.ShapeDtypeStruct((N_EDGES, H), jnp.float32),
               jax.ShapeDtypeStruct((N_EDGES, 64), jnp.float32)],
)


def _tcs_body(b42_ref, cm_ref, w1, wt, w2s, w2t, s_ref):
    b42 = b42_ref[...]
    cm = cm_ref[...]
    sb8 = _mm(b42, w1[...])
    tp = _mm(b42, wt[...])
    tb8 = cm[:, 0:1] * tp[:, 0:8]
    for m in range(1, S):
        tb8 = tb8 + cm[:, m:m + 1] * tp[:, 8 * m:8 * m + 8]
    s_ref[...] = _mm(sb8, w2s[...]) * _mm(tb8, w2t[...])


_tcs_call = pl.pallas_call(
    _tcs_body,
    grid=(N_TRIP // _TE,),
    in_specs=[_row(_TE, 48), _row(_TE, 8), _full((48, 8)), _full((48, 56)),
              _full((8, 64)), _full((8, 64))],
    out_specs=_row(_TE, 64),
    out_shape=jax.ShapeDtypeStruct((N_TRIP, 64), jnp.float32),
)


def _res_block(e, w1, b1, w2, b2):
    return e + _swish(_mm(_swish(_mm(e, w1[...]) + b1[...]), w2[...]) + b2[...])


def _tcg_body(want_e2, agg_ref, xji_ref, e1o_ref, rbf_ref,
              wup, bw1, bb1, bw2, bb2, wmid, bmid,
              aw1, ab1, aw2, ab2, aw3, ab3, aw4, ab4, wrbf, *out_refs):
    e1n = xji_ref[...] + _swish(_mm(agg_ref[...], wup[...]))
    e1n = _res_block(e1n, bw1, bb1, bw2, bb2)
    e1n = _swish(_mm(e1n, wmid[...]) + bmid[...]) + e1o_ref[...]
    e1n = _res_block(e1n, aw1, ab1, aw2, ab2)
    e1n = _res_block(e1n, aw3, ab3, aw4, ab4)
    out_refs[0][...] = e1n
    if want_e2:
        out_refs[1][...] = _mm(rbf_ref[...], wrbf[...]) * e1n


def _make_tcg(want_e2):
    outs = [jax.ShapeDtypeStruct((N_EDGES, H), jnp.float32)]
    ospecs = [_row(_TE, H)]
    if want_e2:
        outs.append(jax.ShapeDtypeStruct((N_EDGES, H), jnp.float32))
        ospecs.append(_row(_TE, H))
    wspecs = ([_full((64, H))] + [_full((H, H)), _full((1, H))] * 2
              + [_full((H, H)), _full((1, H))] * 5 + [_full((8, H))])
    return pl.pallas_call(
        functools.partial(_tcg_body, want_e2),
        grid=(N_EDGES // _TE,),
        in_specs=[_row(_TE, 64), _row(_TE, H), _row(_TE, H), _row(_TE, 8)]
        + wspecs,
        out_specs=ospecs,
        out_shape=outs,
    )


_tcg_call = _make_tcg(False)
_tcg_e2_call = _make_tcg(True)


def _tci_body(xi_ref, xj_ref, rbf_ref, wr0, br0, w1, w2, w3, bcat, e1_ref):
    rbf0 = _swish(_mm(rbf_ref[...], wr0[...]) + br0[...])
    e1_ref[...] = _swish(_mm(xi_ref[...], w1[...]) + _mm(xj_ref[...], w2[...])
                         + _mm(rbf0, w3[...]) + bcat[...])


_tci_call = pl.pallas_call(
    _tci_body,
    grid=(N_EDGES // _TE,),
    in_specs=[_row(_TE, H), _row(_TE, H), _row(_TE, 8), _full((8, H)),
              _full((1, H)), _full((H, H)), _full((H, H)), _full((H, H)),
              _full((1, H))],
    out_specs=_row(_TE, H),
    out_shape=jax.ShapeDtypeStruct((N_EDGES, H), jnp.float32),
)


_TV = 1024


def _tcv_body(va_ref, vb_ref, b_ref, wu, bu, l1w, l1b, l2w, l2b, wf, out_ref):
    v = _swish(_mm(va_ref[...] + vb_ref[...], wu[...]) + bu[...])
    v = _swish(_mm(v, l1w[...]) + l1b[...])
    v = _swish(_mm(v, l2w[...]) + l2b[...])
    v = v @ wf[...]
    onehot = (b_ref[...][:, None]
              == lax.broadcasted_iota(jnp.int32, (1, N_GRAPHS), 1)
              ).astype(jnp.float32)
    acc = lax.dot_general(v, onehot, (((0,), (0,)), ((), ())))

    @pl.when(pl.program_id(0) == 0)
    def _():
        out_ref[...] = jnp.zeros_like(out_ref)
    out_ref[...] += acc


_tcv_call = pl.pallas_call(
    _tcv_body,
    grid=(_NROWS // _TV,),
    in_specs=[_row(_TV, H), _row(_TV, H),
              pl.BlockSpec((_TV,), lambda i: (i,)), _full((H, OUT_EMB)),
              _full((1, OUT_EMB)), _full((OUT_EMB, OUT_EMB)),
              _full((1, OUT_EMB)), _full((OUT_EMB, OUT_EMB)),
              _full((1, OUT_EMB)), _full((OUT_EMB, OUT_DIM))],
    out_specs=pl.BlockSpec((1, N_GRAPHS), lambda i: (0, 0)),
    out_shape=jax.ShapeDtypeStruct((1, N_GRAPHS), jnp.float32),
)


def _envelope(x):
    p = P_ENV + 1
    a = -(p + 1) * (p + 2) / 2.0
    b = p * (p + 2)
    c = -p * (p + 1) / 2.0
    x4 = (x * x) * (x * x)
    return 1.0 / x + a * x4 * x + b * x4 * x * x + c * x4 * x * x * x


def _dist_emb(dist):
    x = jnp.clip(dist / CUTOFF, 1e-4, None)
    freqs = jnp.arange(1, R + 1, dtype=jnp.float32) * math.pi
    return _envelope(x)[:, None] * jnp.sin(freqs[None, :] * x[:, None])


def _sph_jl_all(x):
    out = []
    for l in range(S):
        z = jnp.clip(x[l], 0.1, None)
        sz = jnp.sin(z)
        cz = jnp.cos(z)
        j0 = sz / z
        if l == 0:
            out.append(j0)
            continue
        j1 = sz / (z * z) - cz / z
        jm, jc = j0, j1
        for ll in range(2, l + 1):
            jm, jc = jc, (2.0 * ll - 1.0) / z * jc - jm
        out.append(jc)
    return out


def _base42(dist_t, ct):
    x = jnp.clip(dist_t / CUTOFF, 1e-4, None)
    ps = [jnp.ones_like(ct), ct]
    for l in range(2, S):
        ps.append(((2.0 * l - 1.0) * ct * ps[l - 1] - (l - 1.0) * ps[l - 2]) / l)
    zs = []
    for l in range(S):
        roots = (jnp.arange(1, R + 1, dtype=jnp.float32) + 0.5 * l) * math.pi
        zs.append(roots[None, :] * x[:, None])
    jls = _sph_jl_all(zs)
    feats = [jls[l] * ps[l][:, None] for l in range(S)]
    return jnp.concatenate(feats, axis=1)


def _pad_rows(w, rows):
    return jnp.pad(w, ((0, rows - w.shape[0]), (0, 0)))


def _b2(b):
    return b.reshape(1, -1)


def _update_e(p, e1, rbf8, base48, cosm8, part, want_e2):
    wrc = _pad_rows(p['lin_rbf1']['w'], 8) @ p['lin_rbf2']['w']
    xji, h = _tcf_call(e1, rbf8, p['lin_ji']['w'], _b2(p['lin_ji']['b']),
                       p['lin_kj']['w'], _b2(p['lin_kj']['b']), wrc,
                       p['lin_down']['w'])
    w1 = p['lin_t1']['w'].reshape(S, S, R, BT).transpose(0, 2, 1, 3)
    w1 = _pad_rows(w1.reshape(S * R, S * BT), 48)
    s = _tcs_call(base48, cosm8, _pad_rows(p['lin_sbf1']['w'], 48), w1,
                  p['lin_sbf2']['w'], p['lin_t2']['w'])
    agg = _agg_call(h, s, *part)[:N_EDGES]
    (b1, b2), = p['before_skip']
    (a1, a2), (a3, a4) = p['after_skip']
    wrbf = _pad_rows(p['lin_rbf']['w'], 8)
    call = _tcg_e2_call if want_e2 else _tcg_call
    outs = call(agg, xji, e1, rbf8, p['lin_up']['w'],
                b1['w'], _b2(b1['b']), b2['w'], _b2(b2['b']),
                p['lin_mid']['w'], _b2(p['lin_mid']['b']),
                a1['w'], _b2(a1['b']), a2['w'], _b2(a2['b']),
                a3['w'], _b2(a3['b']), a4['w'], _b2(a4['b']), wrbf)
    if want_e2:
        return outs[0], outs[1]
    return outs[0], None


def kernel(atoms, pos, batch, edge_index, idx_kj, idx_ji, idx_t, params):
    j_idx = edge_index[0]
    i_idx = edge_index[1]
    vecs = pos[j_idx] - pos[i_idx]
    dist = jnp.sqrt(jnp.sum(vecs ** 2, axis=-1) + 1e-12)
    pos_ji = vecs[idx_ji]
    pos_kj = vecs[idx_kj]
    ref_v = vecs[idx_t]
    a = jnp.sum(pos_ji * pos_kj, axis=-1)
    n1 = jnp.cross(pos_ji, pos_kj)
    b = jnp.sqrt(jnp.sum(n1 ** 2, axis=-1) + 1e-12)
    ct = a / jnp.sqrt(a * a + b * b)
    n2 = jnp.cross(pos_ji, ref_v)
    dist_ji = jnp.sqrt(jnp.sum(pos_ji ** 2, axis=-1) + 1e-12)
    t_b = jnp.sum(jnp.cross(n1, n2) * pos_ji, axis=-1) / dist_ji + 1e-6
    t_a = jnp.sum(n1 * n2, axis=-1) + 1e-6
    cphi = t_a / jnp.sqrt(t_a * t_a + t_b * t_b + 1e-30)
    cs = [jnp.ones_like(cphi), cphi]
    for m in range(2, S):
        cs.append(2.0 * cphi * cs[m - 1] - cs[m - 2])
    cosm = jnp.stack(cs, axis=1)

    rbf = _dist_emb(dist)
    dist_t = jnp.sqrt(jnp.sum(pos_kj ** 2, axis=-1) + 1e-12)
    base42 = _base42(dist_t, ct)

    rbf8 = jnp.pad(rbf, ((0, 0), (0, 2)))
    base48 = jnp.pad(base42, ((0, 0), (0, 6)))
    cosm8 = jnp.pad(cosm, ((0, 0), (0, 1)))

    x = params['node_emb'][atoms]
    pi_ = params['init']
    wcat = pi_['lin']['w']
    e1 = _tci_call(x[i_idx], x[j_idx], rbf8, _pad_rows(pi_['rbf0']['w'], 8),
                   _b2(pi_['rbf0']['b']), wcat[:H], wcat[H:2 * H],
                   wcat[2 * H:], _b2(pi_['lin']['b']))

    part = _part_call(idx_ji.astype(jnp.int32), idx_kj.astype(jnp.int32))
    e2 = None
    for layer in range(NUM_LAYERS):
        e1, e2 = _update_e(params['update_es'][layer], e1, rbf8, base48,
                           cosm8, part, want_e2=(layer == NUM_LAYERS - 1))

    pv = params['update_vs'][NUM_LAYERS - 1]
    vp = _seg_nodes_call(e2, i_idx.astype(jnp.int32))
    bpad = jnp.concatenate([batch.astype(jnp.int32),
                            jnp.full((_NROWS - N_NODES,), N_GRAPHS, jnp.int32)])
    out = _tcv_call(vp[0], vp[1], bpad, pv['lin_up']['w'],
                    _b2(pv['lin_up']['b']), pv['lins'][0]['w'],
                    _b2(pv['lins'][0]['b']), pv['lins'][1]['w'],
                    _b2(pv['lins'][1]['b']), pv['lin']['w'])
    return out.reshape(N_GRAPHS, OUT_DIM)
